# baseline (speedup 1.0000x reference)
import functools

import jax
import jax.numpy as jnp
from jax import lax
from jax.experimental import pallas as pl
from jax.experimental.pallas import tpu as pltpu
from jax.experimental.pallas import tpu_sc as plsc

M = 4096
N = 8192
C = 256
H = 128
K = 16


_BKNN = 512


def _knn_body(q_ref, st_ref, qn_ref, kn_ref, idx_ref):
    q = q_ref[:]
    st = st_ref[:]
    cross = lax.dot_general(q, st, (((1,), (0,)), ((), ())),
                            preferred_element_type=jnp.float32)
    d = (qn_ref[:] + kn_ref[:]) - 2.0 * cross
    colid = lax.broadcasted_iota(jnp.int32, (_BKNN, M), 1)
    big = jnp.int32(M)
    for k in range(K):
        m = jnp.min(d, axis=1, keepdims=True)
        sel = jnp.min(jnp.where(d == m, colid, big), axis=1, keepdims=True)
        idx_ref[:, k : k + 1] = sel
        d = jnp.where(colid == sel, jnp.inf, d)


def _knn(dense_coord, sparse_coord_t, qn, kn):
    return pl.pallas_call(
        _knn_body,
        grid=(N // _BKNN,),
        in_specs=[
            pl.BlockSpec((_BKNN, 3), lambda i: (i, 0)),
            pl.BlockSpec((3, M), lambda i: (0, 0)),
            pl.BlockSpec((_BKNN, 1), lambda i: (i, 0)),
            pl.BlockSpec((1, M), lambda i: (0, 0)),
        ],
        out_specs=pl.BlockSpec((_BKNN, K), lambda i: (i, 0)),
        out_shape=jax.ShapeDtypeStruct((N, K), jnp.int32),
    )(dense_coord, sparse_coord_t, qn, kn)


def _proj_body(feat_ref, sp_ref, dn_ref, w1_ref, b1_ref, p_ref, q_ref):
    w1a = w1_ref[0:C, :]
    w1b = w1_ref[C : C + 3, :]
    sp = sp_ref[:]
    dn = dn_ref[:]
    fproj = jnp.dot(feat_ref[:], w1a, preferred_element_type=jnp.float32,
                    precision=lax.Precision.HIGHEST)
    spb = (sp[:, 0:1] * w1b[0:1, :] + sp[:, 1:2] * w1b[1:2, :]
           + sp[:, 2:3] * w1b[2:3, :])
    dnb = (dn[:, 0:1] * w1b[0:1, :] + dn[:, 1:2] * w1b[1:2, :]
           + dn[:, 2:3] * w1b[2:3, :])
    p_ref[:] = fproj - spb
    q_ref[:] = dnb + b1_ref[:]


def _proj(sparse_feat, sparse_coord, dense_coord, W1, b1_row):
    return pl.pallas_call(
        _proj_body,
        out_shape=(
            jax.ShapeDtypeStruct((M, H), jnp.float32),
            jax.ShapeDtypeStruct((N, H), jnp.float32),
        ),
    )(sparse_feat, sparse_coord, dense_coord, W1, b1_row)


_ROWS = N * K
_NC = 2
_NS = 16
_NW = _NC * _NS
_RPW = _ROWS // _NW
_CHUNK = 256
_NCHUNK = _RPW // _CHUNK


def _gather_body(idx_hbm, table_hbm, out_hbm, idx_v, buf0, buf1, sem0, sem1):
    wid = lax.axis_index("s") * _NC + lax.axis_index("c")
    base = wid * _RPW
    pltpu.sync_copy(idx_hbm.at[pl.ds(base, _RPW)], idx_v)
    bufs = (buf0, buf1)
    sems = (sem0, sem1)
    prev = None
    for c in range(_NCHUNK):
        i = c % 2
        cp = pltpu.async_copy(
            table_hbm.at[idx_v.at[pl.ds(c * _CHUNK, _CHUNK)]], bufs[i], sems[i]
        )
        if prev is not None:
            pc, pcp = prev
            pcp.wait()
            pltpu.sync_copy(
                bufs[pc % 2], out_hbm.at[pl.ds(base + pc * _CHUNK, _CHUNK)]
            )
        prev = (c, cp)
    pc, pcp = prev
    pcp.wait()
    pltpu.sync_copy(bufs[pc % 2], out_hbm.at[pl.ds(base + pc * _CHUNK, _CHUNK)])


def _gather(idx_flat, table, dtype):
    mesh = plsc.VectorSubcoreMesh(core_axis_name="c", subcore_axis_name="s")
    fn = functools.partial(
        pl.kernel,
        mesh=mesh,
        out_type=jax.ShapeDtypeStruct((_ROWS, H), dtype),
        scratch_types=[
            pltpu.VMEM((_RPW,), jnp.int32),
            pltpu.VMEM((_CHUNK, H), dtype),
            pltpu.VMEM((_CHUNK, H), dtype),
            pltpu.SemaphoreType.DMA,
            pltpu.SemaphoreType.DMA,
        ],
    )(_gather_body)
    return fn(idx_flat, table)


_BQ = 256


def _score_body(g_ref, q_ref, gam_ref, bet_ref, w2_ref, b2_ref, w_ref):
    g = g_ref[:].reshape(_BQ, K, H)
    h = g + q_ref[:][:, None, :]
    mu = jnp.mean(h, axis=-1, keepdims=True)
    var = jnp.mean((h - mu) ** 2, axis=-1, keepdims=True)
    hn = (h - mu) / jnp.sqrt(var + 1e-5) * gam_ref[:][None] + bet_ref[:][None]
    r = jnp.maximum(hn, 0.0)
    sc = jnp.sum(r * w2_ref[:][None], axis=-1) + b2_ref[0, 0]
    mx = jnp.max(sc, axis=-1, keepdims=True)
    e = jnp.exp(sc - mx)
    w_ref[:] = e / jnp.sum(e, axis=-1, keepdims=True)


def _score(G, Q, gamma_row, beta_row, w2_row, b2_mat):
    return pl.pallas_call(
        _score_body,
        grid=(N // _BQ,),
        in_specs=[
            pl.BlockSpec((_BQ * K, H), lambda i: (i, 0)),
            pl.BlockSpec((_BQ, H), lambda i: (i, 0)),
            pl.BlockSpec((1, H), lambda i: (0, 0)),
            pl.BlockSpec((1, H), lambda i: (0, 0)),
            pl.BlockSpec((1, H), lambda i: (0, 0)),
            pl.BlockSpec((1, 1), lambda i: (0, 0)),
        ],
        out_specs=pl.BlockSpec((_BQ, K), lambda i: (i, 0)),
        out_shape=jax.ShapeDtypeStruct((N, K), jnp.float32),
    )(G, Q, gamma_row, beta_row, w2_row, b2_mat)


def _wsum_body(f_ref, w_ref, out_ref):
    f = f_ref[:].reshape(_BQ, K, C)
    w = w_ref[:]
    acc = f[:, 0, :] * w[:, 0:1]
    for k in range(1, K):
        acc = acc + f[:, k, :] * w[:, k : k + 1]
    out_ref[:] = acc


def _wsum(F, w):
    return pl.pallas_call(
        _wsum_body,
        grid=(N // _BQ,),
        in_specs=[
            pl.BlockSpec((_BQ * K, C), lambda i: (i, 0)),
            pl.BlockSpec((_BQ, K), lambda i: (i, 0)),
        ],
        out_specs=pl.BlockSpec((_BQ, C), lambda i: (i, 0)),
        out_shape=jax.ShapeDtypeStruct((N, C), jnp.float32),
    )(F, w)


@jax.jit
def kernel(sparse_coord, sparse_feat, sparse_offset, dense_coord, dense_offset,
           W1, b1, gamma, beta, W2, b2):
    qn = jnp.sum(dense_coord ** 2, axis=1, keepdims=True)
    kn = jnp.sum(sparse_coord ** 2, axis=1).reshape(1, M)
    idx = _knn(dense_coord, sparse_coord.T, qn, kn)
    P, Q = _proj(sparse_feat, sparse_coord, dense_coord, W1,
                 b1.reshape(1, H))
    idx_flat = idx.reshape(_ROWS)
    G = _gather(idx_flat, P, jnp.float32)
    feat_i32 = lax.bitcast_convert_type(
        sparse_feat.astype(jnp.bfloat16).reshape(M, H, 2), jnp.int32)
    F_i32 = _gather(idx_flat, feat_i32, jnp.int32)
    w = _score(G, Q, gamma.reshape(1, H), beta.reshape(1, H),
               W2.reshape(1, H), b2.reshape(1, 1))
    F = lax.bitcast_convert_type(F_i32, jnp.bfloat16).reshape(_ROWS, C)
    return _wsum(F, w)


# speedup vs baseline: 1.5758x; 1.5758x over previous
import functools

import jax
import jax.numpy as jnp
from jax import lax
from jax.experimental import pallas as pl
from jax.experimental.pallas import tpu as pltpu
from jax.experimental.pallas import tpu_sc as plsc

M = 4096
N = 8192
C = 256
H = 128
K = 16


_BKNN = 512


def _knn_body(q_ref, st_ref, qn_ref, kn_ref, idx_ref):
    q = q_ref[:]
    st = st_ref[:]
    cross = lax.dot_general(q, st, (((1,), (0,)), ((), ())),
                            preferred_element_type=jnp.float32)
    d = (qn_ref[:] + kn_ref[:]) - 2.0 * cross
    colid = lax.broadcasted_iota(jnp.int32, (_BKNN, M), 1)
    big = jnp.int32(M)
    for k in range(K):
        m = jnp.min(d, axis=1, keepdims=True)
        sel = jnp.min(jnp.where(d == m, colid, big), axis=1, keepdims=True)
        idx_ref[:, k : k + 1] = sel
        d = jnp.where(colid == sel, jnp.inf, d)


def _knn(dense_coord, sparse_coord_t, qn, kn):
    return pl.pallas_call(
        _knn_body,
        grid=(N // _BKNN,),
        in_specs=[
            pl.BlockSpec((_BKNN, 3), lambda i: (i, 0)),
            pl.BlockSpec((3, M), lambda i: (0, 0)),
            pl.BlockSpec((_BKNN, 1), lambda i: (i, 0)),
            pl.BlockSpec((1, M), lambda i: (0, 0)),
        ],
        out_specs=pl.BlockSpec((_BKNN, K), lambda i: (i, 0)),
        out_shape=jax.ShapeDtypeStruct((N, K), jnp.int32),
    )(dense_coord, sparse_coord_t, qn, kn)


def _proj_body(feat_ref, sp_ref, dn_ref, w1_ref, b1_ref, p_ref, q_ref):
    w1a = w1_ref[0:C, :]
    w1b = w1_ref[C : C + 3, :]
    sp = sp_ref[:]
    dn = dn_ref[:]
    fproj = jnp.dot(feat_ref[:], w1a, preferred_element_type=jnp.float32,
                    precision=lax.Precision.HIGHEST)
    spb = (sp[:, 0:1] * w1b[0:1, :] + sp[:, 1:2] * w1b[1:2, :]
           + sp[:, 2:3] * w1b[2:3, :])
    dnb = (dn[:, 0:1] * w1b[0:1, :] + dn[:, 1:2] * w1b[1:2, :]
           + dn[:, 2:3] * w1b[2:3, :])
    p_ref[:] = fproj - spb
    q_ref[:] = dnb + b1_ref[:]


def _proj(sparse_feat, sparse_coord, dense_coord, W1, b1_row):
    return pl.pallas_call(
        _proj_body,
        out_shape=(
            jax.ShapeDtypeStruct((M, H), jnp.float32),
            jax.ShapeDtypeStruct((N, H), jnp.float32),
        ),
    )(sparse_feat, sparse_coord, dense_coord, W1, b1_row)


_ROWS = N * K
_NC = 2
_NS = 16
_NW = _NC * _NS
_RPW = _ROWS // _NW
_CHUNK = 256
_NCHUNK = _RPW // _CHUNK


def _gather_body(idx_hbm, table_hbm, out_hbm, idx_v, buf0, buf1, sem0, sem1):
    wid = lax.axis_index("s") * _NC + lax.axis_index("c")
    base = wid * _RPW
    pltpu.sync_copy(idx_hbm.at[pl.ds(base, _RPW)], idx_v)
    bufs = (buf0, buf1)
    sems = (sem0, sem1)
    prev = None
    for c in range(_NCHUNK):
        i = c % 2
        cp = pltpu.async_copy(
            table_hbm.at[idx_v.at[pl.ds(c * _CHUNK, _CHUNK)]], bufs[i], sems[i]
        )
        if prev is not None:
            pc, pcp = prev
            pcp.wait()
            pltpu.sync_copy(
                bufs[pc % 2], out_hbm.at[pl.ds(base + pc * _CHUNK, _CHUNK)]
            )
        prev = (c, cp)
    pc, pcp = prev
    pcp.wait()
    pltpu.sync_copy(bufs[pc % 2], out_hbm.at[pl.ds(base + pc * _CHUNK, _CHUNK)])


def _gather(idx_flat, table, dtype):
    mesh = plsc.VectorSubcoreMesh(core_axis_name="c", subcore_axis_name="s")
    fn = functools.partial(
        pl.kernel,
        mesh=mesh,
        out_type=jax.ShapeDtypeStruct((_ROWS, H), dtype),
        scratch_types=[
            pltpu.VMEM((_RPW,), jnp.int32),
            pltpu.VMEM((_CHUNK, H), dtype),
            pltpu.VMEM((_CHUNK, H), dtype),
            pltpu.SemaphoreType.DMA,
            pltpu.SemaphoreType.DMA,
        ],
    )(_gather_body)
    return fn(idx_flat, table)


_BQ = 256


def _score_body(g_ref, q_ref, gam_ref, bet_ref, w2_ref, b2_ref, w_ref):
    g = g_ref[:].reshape(_BQ, K, H)
    h = g + q_ref[:][:, None, :]
    mu = jnp.mean(h, axis=-1, keepdims=True)
    var = jnp.mean((h - mu) ** 2, axis=-1, keepdims=True)
    hn = (h - mu) / jnp.sqrt(var + 1e-5) * gam_ref[:][None] + bet_ref[:][None]
    r = jnp.maximum(hn, 0.0)
    sc = jnp.sum(r * w2_ref[:][None], axis=-1) + b2_ref[0, 0]
    mx = jnp.max(sc, axis=-1, keepdims=True)
    e = jnp.exp(sc - mx)
    w_ref[:] = e / jnp.sum(e, axis=-1, keepdims=True)


def _score(G, Q, gamma_row, beta_row, w2_row, b2_mat):
    return pl.pallas_call(
        _score_body,
        grid=(N // _BQ,),
        in_specs=[
            pl.BlockSpec((_BQ * K, H), lambda i: (i, 0)),
            pl.BlockSpec((_BQ, H), lambda i: (i, 0)),
            pl.BlockSpec((1, H), lambda i: (0, 0)),
            pl.BlockSpec((1, H), lambda i: (0, 0)),
            pl.BlockSpec((1, H), lambda i: (0, 0)),
            pl.BlockSpec((1, 1), lambda i: (0, 0)),
        ],
        out_specs=pl.BlockSpec((_BQ, K), lambda i: (i, 0)),
        out_shape=jax.ShapeDtypeStruct((N, K), jnp.float32),
    )(G, Q, gamma_row, beta_row, w2_row, b2_mat)


def _wsum_body(f_ref, w_ref, out_ref):
    fi = f_ref[:].reshape(_BQ, K, H)
    w = w_ref[:]
    himask = jnp.int32(-65536)
    acc_lo = None
    acc_hi = None
    for k in range(K):
        x = fi[:, k, :]
        lo = lax.bitcast_convert_type(lax.shift_left(x, 16), jnp.float32)
        hi = lax.bitcast_convert_type(x & himask, jnp.float32)
        wk = w[:, k : k + 1]
        acc_lo = lo * wk if acc_lo is None else acc_lo + lo * wk
        acc_hi = hi * wk if acc_hi is None else acc_hi + hi * wk
    out_ref[:, 0:H] = acc_lo
    out_ref[:, H : 2 * H] = acc_hi


def _wsum(F_packed, w):
    return pl.pallas_call(
        _wsum_body,
        grid=(N // _BQ,),
        in_specs=[
            pl.BlockSpec((_BQ * K, H), lambda i: (i, 0)),
            pl.BlockSpec((_BQ, K), lambda i: (i, 0)),
        ],
        out_specs=pl.BlockSpec((_BQ, C), lambda i: (i, 0)),
        out_shape=jax.ShapeDtypeStruct((N, C), jnp.float32),
    )(F_packed, w)


@jax.jit
def kernel(sparse_coord, sparse_feat, sparse_offset, dense_coord, dense_offset,
           W1, b1, gamma, beta, W2, b2):
    qn = jnp.sum(dense_coord ** 2, axis=1, keepdims=True)
    kn = jnp.sum(sparse_coord ** 2, axis=1).reshape(1, M)
    idx = _knn(dense_coord, sparse_coord.T, qn, kn)
    P, Q = _proj(sparse_feat, sparse_coord, dense_coord, W1,
                 b1.reshape(1, H))
    idx_flat = idx.reshape(_ROWS)
    G = _gather(idx_flat, P, jnp.float32)
    fb = sparse_feat.astype(jnp.bfloat16)
    lo16 = lax.bitcast_convert_type(fb[:, :H], jnp.uint16).astype(jnp.uint32)
    hi16 = lax.bitcast_convert_type(fb[:, H:], jnp.uint16).astype(jnp.uint32)
    feat_packed = lax.bitcast_convert_type(lo16 | (hi16 << 16), jnp.int32)
    F_packed = _gather(idx_flat, feat_packed, jnp.int32)
    w = _score(G, Q, gamma.reshape(1, H), beta.reshape(1, H),
               W2.reshape(1, H), b2.reshape(1, 1))
    return _wsum(F_packed, w)
